# baseline (speedup 1.0000x reference)
import jax
import jax.numpy as jnp
from jax.experimental import pallas as pl
from jax.experimental.pallas import tpu as pltpu

_NUM_CLASSES = 100000
_BLOCK_ROWS = 32


def _onehot_block(idx_ref, out_ref):
    idx = idx_ref[:]
    cols = jax.lax.broadcasted_iota(jnp.int32, (_BLOCK_ROWS, _NUM_CLASSES), 1)
    out_ref[:, :] = (idx == cols).astype(jnp.float32)


def kernel(idx):
    b = idx.shape[0]
    idx2 = idx.astype(jnp.int32).reshape(b, 1)
    grid = (b // _BLOCK_ROWS,)
    return pl.pallas_call(
        _onehot_block,
        grid=grid,
        in_specs=[pl.BlockSpec((_BLOCK_ROWS, 1), lambda i: (i, 0))],
        out_specs=pl.BlockSpec((_BLOCK_ROWS, _NUM_CLASSES), lambda i: (i, 0)),
        out_shape=jax.ShapeDtypeStruct((b, _NUM_CLASSES), jnp.float32),
        compiler_params=pltpu.CompilerParams(
            dimension_semantics=("parallel",),
        ),
    )(idx2)
